# Initial kernel scaffold; baseline (speedup 1.0000x reference)
import jax
import jax.numpy as jnp
from jax.experimental import pallas as pl


def kernel(x, edge_index, edge_attr, batch, W1, b1, Wp1, bp1, g1, bt1, W2, b2, Wp2, bp2, g2, bt2, Wf, bf):
    raise NotImplementedError("write your pallas kernel here")


# speedup vs baseline: 5.6928x; 5.6928x over previous
import functools

import jax
import jax.numpy as jnp
from jax import lax
from jax.experimental import pallas as pl
from jax.experimental.pallas import tpu as pltpu
from jax.experimental.pallas import tpu_sc as plsc

N = 10000
E = 320000
H = 128
C = 2

NC = 2
NS = 16
NW = NC * NS
EPW = E // NW
CK = 80
NB = 25
NBLK = EPW // (NB * CK)
NPAD = 10240
RPT = NPAD // NS

_mesh = plsc.VectorSubcoreMesh(core_axis_name="c", subcore_axis_name="s")


@functools.partial(
    pl.kernel,
    mesh=_mesh,
    out_type=jax.ShapeDtypeStruct((NC, NS, RPT, H), jnp.float32),
    scratch_types=[
        pltpu.VMEM((NB, CK), jnp.int32),
        pltpu.VMEM((NB, CK), jnp.int32),
        pltpu.VMEM((NB, CK), jnp.float32),
        pltpu.VMEM((CK, H), jnp.float32),
        pltpu.VMEM_SHARED((NPAD, H), jnp.float32),
        pltpu.SemaphoreType.DMA,
    ],
)
def _sc_message_pass(h_hbm, src_hbm, dst_hbm, w_hbm, zero_hbm, out_hbm,
                     srcv, dstv, wv, rows, acc, sem):
    cid = lax.axis_index("c")
    sid = lax.axis_index("s")
    wid = sid * NC + cid

    pltpu.sync_copy(zero_hbm.at[pl.ds(sid * RPT, RPT)],
                    acc.at[pl.ds(sid * RPT, RPT)])
    plsc.subcore_barrier()

    def blk_body(bi, carry):
        pltpu.sync_copy(src_hbm.at[wid, bi], srcv)
        pltpu.sync_copy(dst_hbm.at[wid, bi], dstv)
        pltpu.sync_copy(w_hbm.at[wid, bi], wv)

        def chunk_body(cj, c1):
            pltpu.async_copy(h_hbm.at[srcv.at[cj]], rows, sem).wait()

            def group_body(g, c2):
                w16 = jnp.abs(wv[cj, pl.ds(g * 16, 16)])
                base = g * 16
                for j in range(16):
                    ws = w16[j]
                    for f in range(H // 16):
                        sl = pl.ds(f * 16, 16)
                        rows[base + j, sl] = rows[base + j, sl] * ws
                return c2
            lax.fori_loop(0, CK // 16, group_body, 0)

            pltpu.sync_copy(rows, acc.at[dstv.at[cj]], add=True)
            return c1
        lax.fori_loop(0, NB, chunk_body, 0)
        return carry

    lax.fori_loop(0, NBLK, blk_body, 0)
    plsc.subcore_barrier()

    pltpu.sync_copy(acc.at[pl.ds(sid * RPT, RPT)], out_hbm.at[cid, sid])


def _tc_head(x_ref, w_ref, o_ref):
    o_ref[...] = lax.dot_general(
        x_ref[...], w_ref[...], (((1,), (1,)), ((), ())),
        preferred_element_type=jnp.float32)


def _block_tail(p_ref, b_ref, wp_ref, bp_ref, g_ref, bt_ref):
    agg = p_ref[0] + p_ref[1] + b_ref[...]
    o = lax.dot_general(agg, wp_ref[...], (((1,), (1,)), ((), ())),
                        preferred_element_type=jnp.float32) + bp_ref[...]
    o = jnp.where(o >= 0, o, 0.2 * o)
    mean = jnp.mean(o, axis=0, keepdims=True)
    d = o - mean
    var = jnp.mean(d * d, axis=0, keepdims=True)
    return d * lax.rsqrt(var + 1e-5) * g_ref[...] + bt_ref[...]


def _tc_mid(p_ref, b_ref, wp_ref, bp_ref, g_ref, bt_ref, wn_ref, o_ref):
    z = _block_tail(p_ref, b_ref, wp_ref, bp_ref, g_ref, bt_ref)
    o_ref[...] = lax.dot_general(z, wn_ref[...], (((1,), (1,)), ((), ())),
                                 preferred_element_type=jnp.float32)


def _tc_post(p_ref, b_ref, wp_ref, bp_ref, g_ref, bt_ref, wf_ref, bf_ref,
             o_ref):
    z = _block_tail(p_ref, b_ref, wp_ref, bp_ref, g_ref, bt_ref)
    pooled = jnp.mean(z, axis=0, keepdims=True)
    logits = jnp.sum(pooled * wf_ref[...], axis=1)
    o_ref[...] = logits.reshape(1, C) + bf_ref[...]


def kernel(x, edge_index, edge_attr, batch,
           W1, b1, Wp1, bp1, g1, bt1,
           W2, b2, Wp2, bp2, g2, bt2,
           Wf, bf):
    src = edge_index[0].reshape(NW, NBLK, NB, CK)
    dst = edge_index[1].reshape(NW, NBLK, NB, CK)
    w = edge_attr.reshape(NW, NBLK, NB, CK)
    zeros = jnp.zeros((NPAD, H), jnp.float32)

    f32 = jnp.float32
    mm = pl.pallas_call(
        _tc_head, out_shape=jax.ShapeDtypeStruct((N, H), f32))
    mid = pl.pallas_call(
        _tc_mid, out_shape=jax.ShapeDtypeStruct((N, H), f32))
    post = pl.pallas_call(
        _tc_post, out_shape=jax.ShapeDtypeStruct((1, C), f32))

    h1 = mm(x, W1)
    p1 = _sc_message_pass(h1, src, dst, w, zeros).reshape(NC, NPAD, H)[:, :N]
    h2 = mid(p1, b1.reshape(1, H), Wp1, bp1.reshape(1, H),
             g1.reshape(1, H), bt1.reshape(1, H), W2)
    p2 = _sc_message_pass(h2, src, dst, w, zeros).reshape(NC, NPAD, H)[:, :N]
    out = post(p2, b2.reshape(1, H), Wp2,
               bp2.reshape(1, H), g2.reshape(1, H), bt2.reshape(1, H),
               Wf, bf.reshape(1, C))
    return out
